# baseline (speedup 1.0000x reference)
import functools

import jax
import jax.numpy as jnp
from jax import lax
from jax.experimental import pallas as pl
from jax.experimental.pallas import tpu as pltpu
from jax.experimental.pallas import tpu_sc as plsc

_B = 4096
_L = 200
_D = 64
_DP = 128
_LANES = 16
_NC = 2
_NS = 16
_NW = _NC * _NS
_SEQ_PER_W = _B // _NW
_ROWS_PER_W = _SEQ_PER_W * _L
_CH = _L
_NCHUNK = _SEQ_PER_W


def _emb_body(x_hbm, tab_hbm, pos_hbm, out_hbm, pos_v, i0, i1,
              r0, r1, s0v, s1v, g0, g1, s0, s1, o0, o1):
    ibuf = (i0, i1)
    rows = (r0, r1)
    st = (s0v, s1v)
    gsem = (g0, g1)
    isem = (s0, s1)
    osem = (o0, o1)
    cid = lax.axis_index("c")
    sid = lax.axis_index("s")
    wid = sid * _NC + cid
    base = wid * _ROWS_PER_W

    pltpu.sync_copy(pos_hbm, pos_v)

    def idxcp(ch, j):
        return pltpu.make_async_copy(
            x_hbm.at[pl.ds(base + ch * _CH, _CH)], ibuf[j], isem[j])

    def gather(ch, j):
        return pltpu.make_async_copy(
            tab_hbm.at[ibuf[j]], rows[j], gsem[j])

    def outcp(ch, j):
        off = pl.multiple_of(base + ch * _CH, 8)
        return pltpu.make_async_copy(
            st[j], out_hbm.at[pl.ds(off, _CH)], osem[j])

    idxcp(0, 0).start()
    idxcp(1, 1).start()
    idxcp(0, 0).wait()
    gather(0, 0).start()
    idxcp(1, 1).wait()
    gather(1, 1).start()

    @pl.loop(0, _NCHUNK, step=2)
    def _(k2):
        for j in range(2):
            ch = k2 + j
            gather(ch, j).wait()

            @pl.when(ch >= 2)
            def _():
                outcp(ch - 2, j).wait()

            @pl.when(ch < _NCHUNK - 2)
            def _():
                idxcp(ch + 2, j).start()

            @pl.loop(0, _CH, step=2)
            def _(r):
                for rr in range(2):
                    for c in range(_D // _LANES):
                        sl = pl.ds(c * _LANES, _LANES)
                        st[j][r + rr, sl] = (rows[j][r + rr, sl]
                                             + pos_v[r + rr, sl])

            outcp(ch, j).start()

            @pl.when(ch < _NCHUNK - 2)
            def _():
                idxcp(ch + 2, j).wait()
                gather(ch + 2, j).start()

    outcp(_NCHUNK - 2, 0).wait()
    outcp(_NCHUNK - 1, 1).wait()


_PAD_BLK = 8000


def _pad_tc(table):
    def body(t_ref, o_ref):
        o_ref[:, 0:_D] = t_ref[...]

    return pl.pallas_call(
        body,
        grid=(table.shape[0] // _PAD_BLK,),
        in_specs=[pl.BlockSpec((_PAD_BLK, _D), lambda i: (i, 0))],
        out_specs=pl.BlockSpec((_PAD_BLK, _DP), lambda i: (i, 0)),
        out_shape=jax.ShapeDtypeStruct((table.shape[0], _DP), jnp.float32),
    )(table)


@jax.jit
def _emb(x_flat, table_padded, pos):
    mesh = plsc.VectorSubcoreMesh(core_axis_name="c", subcore_axis_name="s")
    run = pl.kernel(
        _emb_body,
        out_type=jax.ShapeDtypeStruct((_B * _L, _D), jnp.float32),
        mesh=mesh,
        scratch_types=[
            pltpu.VMEM((_L, _D), jnp.float32),
            pltpu.VMEM((_CH,), jnp.int32),
            pltpu.VMEM((_CH,), jnp.int32),
            pltpu.VMEM((_CH, _DP), jnp.float32),
            pltpu.VMEM((_CH, _DP), jnp.float32),
            pltpu.VMEM((_CH, _D), jnp.float32),
            pltpu.VMEM((_CH, _D), jnp.float32),
        ] + [pltpu.SemaphoreType.DMA for _ in range(6)],
    )
    return run(x_flat, table_padded, pos)


def kernel(x, table, pos_embed):
    x_flat = x.reshape(-1)
    table_padded = _pad_tc(table)
    pos = pos_embed.reshape(_L, _D)
    out = _emb(x_flat, table_padded, pos)
    return out.reshape(_B, _L, _D)


# speedup vs baseline: 1.1314x; 1.1314x over previous
import functools

import jax
import jax.numpy as jnp
from jax import lax
from jax.experimental import pallas as pl
from jax.experimental.pallas import tpu as pltpu
from jax.experimental.pallas import tpu_sc as plsc

_B = 4096
_L = 200
_D = 64
_DP = 128
_LANES = 16
_NC = 2
_NS = 16
_NW = _NC * _NS
_SEQ_PER_W = _B // _NW
_ROWS_PER_W = _SEQ_PER_W * _L
_CH = _L
_NCHUNK = _SEQ_PER_W


def _emb_body(x_hbm, tab_hbm, pos_hbm, out_hbm, pos_v, i0, i1,
              r0, r1, s0v, s1v, g0, g1, s0, s1, o0, o1):
    ibuf = (i0, i1)
    rows = (r0, r1)
    st = (s0v, s1v)
    gsem = (g0, g1)
    isem = (s0, s1)
    osem = (o0, o1)
    cid = lax.axis_index("c")
    sid = lax.axis_index("s")
    wid = sid * _NC + cid
    base = wid * _ROWS_PER_W

    pltpu.sync_copy(pos_hbm, pos_v)

    def idxcp(ch, j):
        return pltpu.make_async_copy(
            x_hbm.at[pl.ds(base + ch * _CH, _CH)], ibuf[j], isem[j])

    def gather(ch, j):
        return pltpu.make_async_copy(
            tab_hbm.at[ibuf[j]], rows[j], gsem[j])

    def outcp(ch, j):
        off = pl.multiple_of(base + ch * _CH, 8)
        return pltpu.make_async_copy(
            st[j], out_hbm.at[pl.ds(off, _CH)], osem[j])

    idxcp(0, 0).start()
    idxcp(1, 1).start()
    idxcp(0, 0).wait()
    gather(0, 0).start()
    idxcp(1, 1).wait()
    gather(1, 1).start()

    @pl.loop(0, _NCHUNK, step=2)
    def _(k2):
        for j in range(2):
            ch = k2 + j
            gather(ch, j).wait()

            @pl.when(ch >= 2)
            def _():
                outcp(ch - 2, j).wait()

            @pl.when(ch < _NCHUNK - 2)
            def _():
                idxcp(ch + 2, j).start()

            @pl.loop(0, _CH, step=4)
            def _(r):
                for rr in range(4):
                    for c in range(_D // _LANES):
                        sl = pl.ds(c * _LANES, _LANES)
                        st[j][r + rr, sl] = (rows[j][r + rr, sl]
                                             + pos_v[r + rr, sl])

            outcp(ch, j).start()

            @pl.when(ch < _NCHUNK - 2)
            def _():
                idxcp(ch + 2, j).wait()
                gather(ch + 2, j).start()

    outcp(_NCHUNK - 2, 0).wait()
    outcp(_NCHUNK - 1, 1).wait()


@jax.jit
def _emb(x_flat, table_padded, pos):
    mesh = plsc.VectorSubcoreMesh(core_axis_name="c", subcore_axis_name="s")
    run = pl.kernel(
        _emb_body,
        out_type=jax.ShapeDtypeStruct((_B * _L, _D), jnp.float32),
        mesh=mesh,
        scratch_types=[
            pltpu.VMEM((_L, _D), jnp.float32),
            pltpu.VMEM((_CH,), jnp.int32),
            pltpu.VMEM((_CH,), jnp.int32),
            pltpu.VMEM((_CH, _DP), jnp.float32),
            pltpu.VMEM((_CH, _DP), jnp.float32),
            pltpu.VMEM((_CH, _D), jnp.float32),
            pltpu.VMEM((_CH, _D), jnp.float32),
        ] + [pltpu.SemaphoreType.DMA for _ in range(6)],
    )
    return run(x_flat, table_padded, pos)


def kernel(x, table, pos_embed):
    x_flat = x.reshape(-1)
    table_padded = jnp.pad(table, ((0, 0), (0, _DP - _D)))
    pos = pos_embed.reshape(_L, _D)
    out = _emb(x_flat, table_padded, pos)
    return out.reshape(_B, _L, _D)
